# baseline (speedup 1.0000x reference)
import functools

import jax
import jax.numpy as jnp
from jax import lax
from jax.experimental import pallas as pl
from jax.experimental.pallas import tpu as pltpu
from jax.experimental.pallas import tpu_sc as plsc

N_EXPERTS = 64
TOP_K = 8
D_MODEL = 4096
TOKENS = 16384

BT = 512
CHUNK_SIZES = (12800, 3584)

NC = 2
NS = 16
L = 16
NW = NC * NS


def _dense_block(x_ref, w_ref, b_ref, s_ref):
    s_ref[...] = jax.nn.sigmoid(
        jnp.dot(x_ref[...], w_ref[...], preferred_element_type=jnp.float32)
        + b_ref[...]
    )


def _tc_scores(x, W, b2, start, ct):
    nb = ct // BT
    base = start // BT
    return pl.pallas_call(
        _dense_block,
        grid=(nb,),
        in_specs=[
            pl.BlockSpec((BT, D_MODEL), lambda i: (base + i, 0)),
            pl.BlockSpec((D_MODEL, N_EXPERTS), lambda i: (0, 0)),
            pl.BlockSpec((1, N_EXPERTS), lambda i: (0, 0)),
        ],
        out_specs=pl.BlockSpec((BT, N_EXPERTS), lambda i: (i, 0)),
        out_shape=jax.ShapeDtypeStruct((ct, N_EXPERTS), jnp.float32),
    )(x, W, b2)


def _merge_tops(ka, ia, kb, ib):
    krb = lax.rev(kb, dimensions=(0,))
    irb = lax.rev(ib, dimensions=(0,))
    take_a = ka >= krb
    return jnp.where(take_a, ka, krb), jnp.where(take_a, ia, irb)


def _sc_topk_body(tpw, s_hbm, gs_hbm, gi_hbm, s_v, gs_v, gi_v):
    wid = lax.axis_index("s") * NC + lax.axis_index("c")
    base = wid * tpw
    pltpu.sync_copy(s_hbm.at[pl.ds(base, tpw)], s_v)

    iota = lax.iota(jnp.int32, L)
    lane_lt8 = iota < TOP_K

    @plsc.parallel_loop(0, tpw, step=1, unroll=4)
    def _token_loop(t):
        k0, i0 = plsc.sort_key_val(s_v[t, pl.ds(0, L)], iota, descending=True)
        k1, i1 = plsc.sort_key_val(s_v[t, pl.ds(L, L)], iota + L, descending=True)
        k2, i2 = plsc.sort_key_val(s_v[t, pl.ds(2 * L, L)], iota + 2 * L, descending=True)
        k3, i3 = plsc.sort_key_val(s_v[t, pl.ds(3 * L, L)], iota + 3 * L, descending=True)
        ek, ei = _merge_tops(k0, i0, k1, i1)
        fk, fi = _merge_tops(k2, i2, k3, i3)
        ek, ei = plsc.sort_key_val(ek, ei, descending=True)
        fk, fi = plsc.sort_key_val(fk, fi, descending=True)
        gk, gi = _merge_tops(ek, ei, fk, fi)
        gk, gi = plsc.sort_key_val(gk, gi, descending=True)
        total = jnp.sum(jnp.where(lane_lt8, gk, 0.0))
        gs_v[t, :] = gk / total
        gi_v[t, :] = gi

    pltpu.sync_copy(gs_v, gs_hbm.at[pl.ds(base, tpw)])
    pltpu.sync_copy(gi_v, gi_hbm.at[pl.ds(base, tpw)])


@functools.cache
def _make_sc_topk(ct):
    tpw = ct // NW
    return pl.kernel(
        functools.partial(_sc_topk_body, tpw),
        out_type=[
            jax.ShapeDtypeStruct((ct, L), jnp.float32),
            jax.ShapeDtypeStruct((ct, L), jnp.int32),
        ],
        mesh=plsc.VectorSubcoreMesh(
            core_axis_name="c", subcore_axis_name="s", num_cores=NC, num_subcores=NS
        ),
        scratch_types=[
            pltpu.VMEM((tpw, N_EXPERTS), jnp.float32),
            pltpu.VMEM((tpw, L), jnp.float32),
            pltpu.VMEM((tpw, L), jnp.int32),
        ],
        compiler_params=pltpu.CompilerParams(
            needs_layout_passes=False, use_tc_tiling_on_sc=False
        ),
    )


@jax.jit
def kernel(x, W, b):
    b2 = b.reshape(1, N_EXPERTS)
    s_chunks = []
    gs_chunks = []
    gi_chunks = []
    start = 0
    for ct in CHUNK_SIZES:
        s_c = _tc_scores(x, W, b2, start, ct)
        gs_c, gi_c = _make_sc_topk(ct)(s_c)
        s_chunks.append(s_c)
        gs_chunks.append(gs_c[:, :TOP_K])
        gi_chunks.append(gi_c[:, :TOP_K])
        start += ct
    s = jnp.concatenate(s_chunks, axis=0)
    gs = jnp.concatenate(gs_chunks, axis=0)
    gi = jnp.concatenate(gi_chunks, axis=0)
    return (gs, gi, s)


# speedup vs baseline: 1.0902x; 1.0902x over previous
import functools

import jax
import jax.numpy as jnp
from jax import lax
from jax.experimental import pallas as pl
from jax.experimental.pallas import tpu as pltpu
from jax.experimental.pallas import tpu_sc as plsc

N_EXPERTS = 64
TOP_K = 8
D_MODEL = 4096
TOKENS = 16384

BT = 512
CHUNK_SIZES = (TOKENS,)

NC = 2
NS = 16
L = 16
NW = NC * NS


def _dense_block(x_ref, w_ref, b_ref, s_ref):
    s_ref[...] = jax.nn.sigmoid(
        jnp.dot(x_ref[...], w_ref[...], preferred_element_type=jnp.float32)
        + b_ref[...]
    )


def _tc_scores(x, W, b2, start, ct):
    nb = ct // BT
    base = start // BT
    return pl.pallas_call(
        _dense_block,
        grid=(nb,),
        in_specs=[
            pl.BlockSpec((BT, D_MODEL), lambda i: (base + i, 0)),
            pl.BlockSpec((D_MODEL, N_EXPERTS), lambda i: (0, 0)),
            pl.BlockSpec((1, N_EXPERTS), lambda i: (0, 0)),
        ],
        out_specs=pl.BlockSpec((BT, N_EXPERTS), lambda i: (i, 0)),
        out_shape=jax.ShapeDtypeStruct((ct, N_EXPERTS), jnp.float32),
    )(x, W, b2)


def _merge_tops(ka, ia, kb, ib):
    krb = lax.rev(kb, dimensions=(0,))
    irb = lax.rev(ib, dimensions=(0,))
    take_a = ka >= krb
    return jnp.where(take_a, ka, krb), jnp.where(take_a, ia, irb)


def _sc_topk_body(tpw, s_hbm, gs_hbm, gi_hbm, s_v, gs_v, gi_v):
    wid = lax.axis_index("s") * NC + lax.axis_index("c")
    base = wid * tpw
    pltpu.sync_copy(s_hbm.at[pl.ds(base, tpw)], s_v)

    iota = lax.iota(jnp.int32, L)
    lane_lt8 = iota < TOP_K

    @plsc.parallel_loop(0, tpw, step=1, unroll=4)
    def _token_loop(t):
        k0, i0 = plsc.sort_key_val(s_v[t, pl.ds(0, L)], iota, descending=True)
        k1, i1 = plsc.sort_key_val(s_v[t, pl.ds(L, L)], iota + L, descending=True)
        k2, i2 = plsc.sort_key_val(s_v[t, pl.ds(2 * L, L)], iota + 2 * L, descending=True)
        k3, i3 = plsc.sort_key_val(s_v[t, pl.ds(3 * L, L)], iota + 3 * L, descending=True)
        ek, ei = _merge_tops(k0, i0, k1, i1)
        fk, fi = _merge_tops(k2, i2, k3, i3)
        ek, ei = plsc.sort_key_val(ek, ei, descending=True)
        fk, fi = plsc.sort_key_val(fk, fi, descending=True)
        gk, gi = _merge_tops(ek, ei, fk, fi)
        gk, gi = plsc.sort_key_val(gk, gi, descending=True)
        total = jnp.sum(jnp.where(lane_lt8, gk, 0.0))
        gs_v[t, :] = gk / total
        gi_v[t, :] = gi

    pltpu.sync_copy(gs_v, gs_hbm.at[pl.ds(base, tpw)])
    pltpu.sync_copy(gi_v, gi_hbm.at[pl.ds(base, tpw)])


@functools.cache
def _make_sc_topk(ct):
    tpw = ct // NW
    return pl.kernel(
        functools.partial(_sc_topk_body, tpw),
        out_type=[
            jax.ShapeDtypeStruct((ct, L), jnp.float32),
            jax.ShapeDtypeStruct((ct, L), jnp.int32),
        ],
        mesh=plsc.VectorSubcoreMesh(
            core_axis_name="c", subcore_axis_name="s", num_cores=NC, num_subcores=NS
        ),
        scratch_types=[
            pltpu.VMEM((tpw, N_EXPERTS), jnp.float32),
            pltpu.VMEM((tpw, L), jnp.float32),
            pltpu.VMEM((tpw, L), jnp.int32),
        ],
        compiler_params=pltpu.CompilerParams(
            needs_layout_passes=False, use_tc_tiling_on_sc=False
        ),
    )


@jax.jit
def kernel(x, W, b):
    b2 = b.reshape(1, N_EXPERTS)
    s_chunks = []
    gs_chunks = []
    gi_chunks = []
    start = 0
    for ct in CHUNK_SIZES:
        s_c = _tc_scores(x, W, b2, start, ct)
        gs_c, gi_c = _make_sc_topk(ct)(s_c)
        s_chunks.append(s_c)
        gs_chunks.append(gs_c[:, :TOP_K])
        gi_chunks.append(gi_c[:, :TOP_K])
        start += ct
    if len(CHUNK_SIZES) == 1:
        return (gs_chunks[0], gi_chunks[0], s_chunks[0])
    s = jnp.concatenate(s_chunks, axis=0)
    gs = jnp.concatenate(gs_chunks, axis=0)
    gi = jnp.concatenate(gi_chunks, axis=0)
    return (gs, gi, s)
